# baseline (speedup 1.0000x reference)
import jax
import jax.numpy as jnp
from jax import lax
from jax.experimental import pallas as pl
from jax.experimental.pallas import tpu as pltpu
from jax.experimental.pallas import tpu_sc as plsc

VOCAB = 1000000
EMBED_DIM = 64
AUX_DIM = 128
TARGET_DIM = 64
BATCH = 16384
HIST = 50
B_TOTAL = BATCH * HIST
HALF_B = BATCH // 2

NC = 2
NS = 16
NW = NC * NS
SEGS = 5
B_SEG = B_TOTAL // SEGS
B_PER_W = B_SEG // NW

GRP = 128
K = 4
CHUNK = GRP * K
N_CHUNKS = B_PER_W // CHUNK
IDX_ROWS_PER_W = B_PER_W // GRP


def _sc_gather_body(idx_hbm, table_hbm, out_hbm,
                    idx_v0, idx_v1, rows_v0, rows_v1, sem0, sem1):
    wid = lax.axis_index("s") * NC + lax.axis_index("c")
    idx_row0 = wid * IDX_ROWS_PER_W
    out_row0 = wid * B_PER_W
    idx_v = (idx_v0, idx_v1)
    rows_v = (rows_v0, rows_v1)
    sems = (sem0, sem1)

    def fire(i, p):
        pltpu.sync_copy(idx_hbm.at[pl.ds(idx_row0 + i * K, K)], idx_v[p])
        for j in range(K):
            pltpu.async_copy(
                table_hbm.at[idx_v[p].at[j]],
                rows_v[p].at[pl.ds(j * GRP, GRP)],
                sems[p],
            )

    def drain_write(i, p):
        for j in range(K):
            pltpu.make_async_copy(
                table_hbm.at[idx_v[p].at[j]],
                rows_v[p].at[pl.ds(j * GRP, GRP)],
                sems[p],
            ).wait()
        pltpu.sync_copy(rows_v[p], out_hbm.at[pl.ds(out_row0 + i * CHUNK, CHUNK)])

    fire(0, 0)
    fire(1, 1)

    def step(j, carry):
        i = 2 * j
        drain_write(i, 0)
        fire(i + 2, 0)
        drain_write(i + 1, 1)
        fire(i + 3, 1)
        return carry

    lax.fori_loop(0, N_CHUNKS // 2 - 1, step, 0)
    i_last = N_CHUNKS - 2
    drain_write(i_last, 0)
    drain_write(i_last + 1, 1)


_sc_gather = pl.kernel(
    _sc_gather_body,
    out_type=jax.ShapeDtypeStruct((B_SEG, EMBED_DIM), jnp.float32),
    mesh=plsc.VectorSubcoreMesh(
        core_axis_name="c", subcore_axis_name="s", num_cores=NC, num_subcores=NS
    ),
    scratch_types=[
        pltpu.VMEM((K, GRP), jnp.int32),
        pltpu.VMEM((K, GRP), jnp.int32),
        pltpu.VMEM((CHUNK, EMBED_DIM), jnp.float32),
        pltpu.VMEM((CHUNK, EMBED_DIM), jnp.float32),
        pltpu.SemaphoreType.DMA,
        pltpu.SemaphoreType.DMA,
    ],
    compiler_params=pltpu.CompilerParams(use_tc_tiling_on_sc=False),
)


PACK_V = 12800
PACK_H = PACK_V // 2
PACK_GRID = -(-VOCAB // PACK_V)
PACK_ROWS = PACK_GRID * PACK_H
VOCAB_PAD = 2 * PACK_ROWS


def _tc_pack_body(t_ref, o_ref):
    x = t_ref[...]
    eye = jnp.eye(EMBED_DIM, dtype=jnp.float32)
    t1 = lax.dot_general(
        x[:, :PACK_H], eye, (((0,), (0,)), ((), ())),
        preferred_element_type=jnp.float32,
    )
    t2 = lax.dot_general(
        x[:, PACK_H:], eye, (((0,), (0,)), ((), ())),
        preferred_element_type=jnp.float32,
    )
    o_ref[...] = jnp.concatenate([t1, t2], axis=1)


_tc_pack = pl.pallas_call(
    _tc_pack_body,
    grid=(PACK_GRID,),
    in_specs=[pl.BlockSpec((EMBED_DIM, PACK_V), lambda i: (0, i))],
    out_specs=pl.BlockSpec((PACK_H, 2 * EMBED_DIM), lambda i: (i, 0)),
    out_shape=jax.ShapeDtypeStruct((PACK_ROWS, 2 * EMBED_DIM), jnp.float32),
)


def _tc_proj_t_body(g_ref, w1_ref, b1_ref, w2_ref, b2d_ref, o_ref, *_unused):
    wct_t = lax.dot_general(
        w2_ref[...], w1_ref[...], (((1,), (0,)), ((), ())),
        preferred_element_type=jnp.float32,
    )
    w2b = w2_ref[...] * b1_ref[...] + b2d_ref[...]
    bias_half = lax.dot_general(
        w2b, jnp.ones((HALF_B, AUX_DIM), jnp.float32), (((1,), (1,)), ((), ())),
        preferred_element_type=jnp.float32,
    )
    z = jnp.zeros((TARGET_DIM, EMBED_DIM), jnp.float32)
    w_full = jnp.concatenate(
        [jnp.concatenate([wct_t, z], axis=1), jnp.concatenate([z, wct_t], axis=1)],
        axis=0,
    )
    res2 = lax.dot_general(
        w_full, g_ref[0], (((1,), (1,)), ((), ())),
        preferred_element_type=jnp.float32,
    )
    res2 = res2 + jnp.concatenate([bias_half, bias_half], axis=0)
    res = jnp.concatenate([res2[:TARGET_DIM], res2[TARGET_DIM:]], axis=1)
    o_ref[...] = res[None]


HIST_SEG = HIST // SEGS


def _make_proj(seg, chained):
    def body(*refs):
        _tc_proj_t_body(*refs[:5], refs[-1])

    in_specs = [
        pl.BlockSpec((1, HALF_B, 2 * EMBED_DIM), lambda i: (i, 0, 0)),
        pl.BlockSpec((AUX_DIM, EMBED_DIM), lambda i: (0, 0)),
        pl.BlockSpec((1, AUX_DIM), lambda i: (0, 0)),
        pl.BlockSpec((TARGET_DIM, AUX_DIM), lambda i: (0, 0)),
        pl.BlockSpec((TARGET_DIM, AUX_DIM), lambda i: (0, 0)),
    ]
    kwargs = {}
    if chained:
        in_specs.append(pl.BlockSpec(memory_space=pltpu.MemorySpace.HBM))
        kwargs["input_output_aliases"] = {5: 0}
    return pl.pallas_call(
        body,
        grid=(HIST_SEG,),
        in_specs=in_specs,
        out_specs=pl.BlockSpec(
            (1, TARGET_DIM, BATCH), lambda i: (i + seg * HIST_SEG, 0, 0)
        ),
        out_shape=jax.ShapeDtypeStruct((HIST, TARGET_DIM, BATCH), jnp.float32),
        **kwargs,
    )


_tc_projs = [_make_proj(s, s > 0) for s in range(SEGS)]


def kernel(indices, table, W1, b1, W2, b2):
    idx_t = indices.astype(jnp.int32).T
    rem = idx_t % PACK_V
    idx_t = 2 * (PACK_H * (idx_t // PACK_V) + rem % PACK_H) + rem // PACK_H
    idx_i = jnp.stack([idx_t[:, :HALF_B], idx_t[:, HALF_B:]], axis=-1)
    idx2d = idx_i.reshape(B_TOTAL // GRP, GRP)
    table_lin = _tc_pack(table.T).reshape(VOCAB_PAD, EMBED_DIM)
    b2d = jnp.broadcast_to((b2 / AUX_DIM).reshape(TARGET_DIM, 1),
                           (TARGET_DIM, AUX_DIM))
    b1r = b1.reshape(1, AUX_DIM)
    seg_rows = B_SEG // GRP
    g_segs = [
        _sc_gather(idx2d[s * seg_rows:(s + 1) * seg_rows], table_lin)
        for s in range(SEGS)
    ]
    out_t = None
    for s in range(SEGS):
        g3 = g_segs[s].reshape(HIST_SEG, HALF_B, 2 * EMBED_DIM)
        if s == 0:
            out_t = _tc_projs[0](g3, W1, b1r, W2, b2d)
        else:
            out_t = _tc_projs[s](g3, W1, b1r, W2, b2d, out_t)
    return jnp.transpose(out_t, (2, 0, 1))


# speedup vs baseline: 1.4618x; 1.4618x over previous
import jax
import jax.numpy as jnp
from jax import lax
from jax.experimental import pallas as pl
from jax.experimental.pallas import tpu as pltpu
from jax.experimental.pallas import tpu_sc as plsc

VOCAB = 1000000
EMBED_DIM = 64
AUX_DIM = 128
TARGET_DIM = 64
BATCH = 16384
HIST = 50
B_TOTAL = BATCH * HIST
HALF_B = BATCH // 2

NC = 2
NS = 16
NW = NC * NS
B_PER_W = B_TOTAL // NW

GRP = 128
K = 4
CHUNK = GRP * K
N_CHUNKS = B_PER_W // CHUNK
IDX_ROWS_PER_W = B_PER_W // GRP


PAIRS_PER_CHUNK = CHUNK // 2
PAIRS_PER_W = B_PER_W // 2


def _sc_gather_body(idx_hbm, table_hbm, out_hbm,
                    idx_a0, idx_a1, idx_b0, idx_b1,
                    idx_v0, idx_v1, rows_v0, rows_v1, sem0, sem1):
    wid = lax.axis_index("s") * NC + lax.axis_index("c")
    out_row0 = wid * B_PER_W
    idx_a = (idx_a0, idx_a1)
    idx_b = (idx_b0, idx_b1)
    idx_v = (idx_v0, idx_v1)
    rows_v = (rows_v0, rows_v1)
    sems = (sem0, sem1)
    lanes = lax.iota(jnp.int32, 16)

    def fire(i, p):
        pair0 = wid * PAIRS_PER_W + i * PAIRS_PER_CHUNK
        l = pair0 // HALF_B
        q0 = (pair0 % HALF_B) // GRP
        pltpu.sync_copy(idx_hbm.at[l, pl.ds(q0, 2)], idx_a[p])
        pltpu.sync_copy(idx_hbm.at[l, pl.ds(q0 + HALF_B // GRP, 2)], idx_b[p])
        half = lanes >> 1
        even = (lanes & 1) == 0
        for t in range(16):
            va = idx_a[p].at[t // 8][pl.ds((t % 8) * 16, 16)]
            vb = idx_b[p].at[t // 8][pl.ds((t % 8) * 16, 16)]
            lo = jnp.where(even, va.at[half].get(mode="promise_in_bounds"),
                           vb.at[half].get(mode="promise_in_bounds"))
            hi = jnp.where(even, va.at[8 + half].get(mode="promise_in_bounds"),
                           vb.at[8 + half].get(mode="promise_in_bounds"))
            idx_v[p].at[t // 4][pl.ds(32 * (t % 4), 16)] = lo
            idx_v[p].at[t // 4][pl.ds(32 * (t % 4) + 16, 16)] = hi
        for j in range(K):
            pltpu.async_copy(
                table_hbm.at[idx_v[p].at[j]],
                rows_v[p].at[pl.ds(j * GRP, GRP)],
                sems[p],
            )

    def drain_write(i, p):
        for j in range(K):
            pltpu.make_async_copy(
                table_hbm.at[idx_v[p].at[j]],
                rows_v[p].at[pl.ds(j * GRP, GRP)],
                sems[p],
            ).wait()
        pltpu.sync_copy(rows_v[p], out_hbm.at[pl.ds(out_row0 + i * CHUNK, CHUNK)])

    fire(0, 0)
    fire(1, 1)

    def step(j, carry):
        i = 2 * j
        drain_write(i, 0)
        fire(i + 2, 0)
        drain_write(i + 1, 1)
        fire(i + 3, 1)
        return carry

    lax.fori_loop(0, N_CHUNKS // 2 - 1, step, 0)
    i_last = N_CHUNKS - 2
    drain_write(i_last, 0)
    drain_write(i_last + 1, 1)


_sc_gather = pl.kernel(
    _sc_gather_body,
    out_type=jax.ShapeDtypeStruct((B_TOTAL, EMBED_DIM), jnp.float32),
    mesh=plsc.VectorSubcoreMesh(
        core_axis_name="c", subcore_axis_name="s", num_cores=NC, num_subcores=NS
    ),
    scratch_types=[
        pltpu.VMEM((2, GRP), jnp.int32),
        pltpu.VMEM((2, GRP), jnp.int32),
        pltpu.VMEM((2, GRP), jnp.int32),
        pltpu.VMEM((2, GRP), jnp.int32),
        pltpu.VMEM((K, GRP), jnp.int32),
        pltpu.VMEM((K, GRP), jnp.int32),
        pltpu.VMEM((CHUNK, EMBED_DIM), jnp.float32),
        pltpu.VMEM((CHUNK, EMBED_DIM), jnp.float32),
        pltpu.SemaphoreType.DMA,
        pltpu.SemaphoreType.DMA,
    ],
    compiler_params=pltpu.CompilerParams(use_tc_tiling_on_sc=False),
)


PACK_V = 12800
PACK_H = PACK_V // 2
PACK_GRID = -(-VOCAB // PACK_V)
PACK_ROWS = PACK_GRID * PACK_H
VOCAB_PAD = 2 * PACK_ROWS


def _tc_pack_body(t_ref, o_ref):
    x = t_ref[...]
    eye = jnp.eye(EMBED_DIM, dtype=jnp.float32)
    t1 = lax.dot_general(
        x[:, :PACK_H], eye, (((0,), (0,)), ((), ())),
        preferred_element_type=jnp.float32,
    )
    t2 = lax.dot_general(
        x[:, PACK_H:], eye, (((0,), (0,)), ((), ())),
        preferred_element_type=jnp.float32,
    )
    o_ref[...] = jnp.concatenate([t1, t2], axis=1)


_tc_pack = pl.pallas_call(
    _tc_pack_body,
    grid=(PACK_GRID,),
    in_specs=[pl.BlockSpec((EMBED_DIM, PACK_V), lambda i: (0, i))],
    out_specs=pl.BlockSpec((PACK_H, 2 * EMBED_DIM), lambda i: (i, 0)),
    out_shape=jax.ShapeDtypeStruct((PACK_ROWS, 2 * EMBED_DIM), jnp.float32),
)


def _tc_proj_t_body(g_ref, w1_ref, b1_ref, w2_ref, b2d_ref, o_ref):
    wct_t = lax.dot_general(
        w2_ref[...], w1_ref[...], (((1,), (0,)), ((), ())),
        preferred_element_type=jnp.float32,
    )
    w2b = w2_ref[...] * b1_ref[...] + b2d_ref[...]
    bias_half = lax.dot_general(
        w2b, jnp.ones((HALF_B, AUX_DIM), jnp.float32), (((1,), (1,)), ((), ())),
        preferred_element_type=jnp.float32,
    )
    z = jnp.zeros((TARGET_DIM, EMBED_DIM), jnp.float32)
    w_full = jnp.concatenate(
        [jnp.concatenate([wct_t, z], axis=1), jnp.concatenate([z, wct_t], axis=1)],
        axis=0,
    )
    res2 = lax.dot_general(
        w_full, g_ref[0], (((1,), (1,)), ((), ())),
        preferred_element_type=jnp.float32,
    )
    res2 = res2 + jnp.concatenate([bias_half, bias_half], axis=0)
    res = jnp.concatenate([res2[:TARGET_DIM], res2[TARGET_DIM:]], axis=1)
    o_ref[...] = res[None]


_tc_proj_t = pl.pallas_call(
    _tc_proj_t_body,
    grid=(HIST,),
    in_specs=[
        pl.BlockSpec((1, HALF_B, 2 * EMBED_DIM), lambda i: (i, 0, 0)),
        pl.BlockSpec((AUX_DIM, EMBED_DIM), lambda i: (0, 0)),
        pl.BlockSpec((1, AUX_DIM), lambda i: (0, 0)),
        pl.BlockSpec((TARGET_DIM, AUX_DIM), lambda i: (0, 0)),
        pl.BlockSpec((TARGET_DIM, AUX_DIM), lambda i: (0, 0)),
    ],
    out_specs=pl.BlockSpec((1, TARGET_DIM, BATCH), lambda i: (i, 0, 0)),
    out_shape=jax.ShapeDtypeStruct((HIST, TARGET_DIM, BATCH), jnp.float32),
)


def kernel(indices, table, W1, b1, W2, b2):
    idx = indices.astype(jnp.int32)
    rem = idx % PACK_V
    ridx = 2 * (PACK_H * (idx // PACK_V) + rem % PACK_H) + rem // PACK_H
    idx3 = ridx.T.reshape(HIST, BATCH // GRP, GRP)
    table_lin = _tc_pack(table.T).reshape(VOCAB_PAD, EMBED_DIM)
    gathered = _sc_gather(idx3, table_lin)
    g3 = gathered.reshape(HIST, HALF_B, 2 * EMBED_DIM)
    b2d = jnp.broadcast_to((b2 / AUX_DIM).reshape(TARGET_DIM, 1),
                           (TARGET_DIM, AUX_DIM))
    out_t = _tc_proj_t(
        g3, W1, b1.reshape(1, AUX_DIM), W2, b2d
    )
    return jnp.transpose(out_t, (2, 0, 1))
